# baseline (speedup 1.0000x reference)
import functools

import jax
import jax.numpy as jnp
from jax import lax
from jax.experimental import pallas as pl
from jax.experimental.pallas import tpu as pltpu
from jax.experimental.pallas import tpu_sc as plsc

CHUNK = 1024


@functools.lru_cache(maxsize=None)
def _build(n_rows: int, vocab: int, dim: int):
    info = plsc.get_sparse_core_info()
    nw = info.num_cores * info.num_subcores
    per_w = n_rows // nw
    assert n_rows % nw == 0 and per_w % CHUNK == 0
    n_chunks = per_w // CHUNK

    mesh = plsc.VectorSubcoreMesh(core_axis_name="c", subcore_axis_name="s")

    @functools.partial(
        pl.kernel,
        mesh=mesh,
        out_type=jax.ShapeDtypeStruct((n_rows, dim), jnp.float32),
        scratch_types=[
            pltpu.VMEM((CHUNK,), jnp.int32),
            pltpu.VMEM((CHUNK, dim), jnp.float32),
            pltpu.SemaphoreType.DMA,
        ],
        compiler_params=pltpu.CompilerParams(use_tc_tiling_on_sc=False),
    )
    def gather_kernel(x_hbm, table_hbm, out_hbm, idx_v, rows_v, sem):
        wid = lax.axis_index("s") * info.num_cores + lax.axis_index("c")
        base = wid * per_w

        def body(i, carry):
            off = base + i * CHUNK
            pltpu.sync_copy(x_hbm.at[pl.ds(off, CHUNK)], idx_v)
            pltpu.async_copy(table_hbm.at[idx_v], rows_v, sem).wait()
            pltpu.sync_copy(rows_v, out_hbm.at[pl.ds(off, CHUNK)])
            return carry

        lax.fori_loop(0, n_chunks, body, 0)

    return gather_kernel


def kernel(x, table):
    n_rows = x.shape[0] * x.shape[1]
    vocab, dim = table.shape
    fn = _build(n_rows, vocab, dim)
    out = fn(x.reshape(-1).astype(jnp.int32), table)
    return out.reshape(x.shape + (dim,))


# speedup vs baseline: 1.0167x; 1.0167x over previous
import functools

import jax
import jax.numpy as jnp
from jax import lax
from jax.experimental import pallas as pl
from jax.experimental.pallas import tpu as pltpu
from jax.experimental.pallas import tpu_sc as plsc

CHUNK = 800


@functools.lru_cache(maxsize=None)
def _build(n_rows: int, vocab: int, dim: int):
    info = plsc.get_sparse_core_info()
    nw = info.num_cores * info.num_subcores
    per_w = n_rows // nw
    assert n_rows % nw == 0 and per_w % CHUNK == 0 and per_w % 8 == 0
    n_chunks = per_w // CHUNK
    assert n_chunks % 2 == 0
    n2 = n_chunks // 2

    mesh = plsc.VectorSubcoreMesh(core_axis_name="c", subcore_axis_name="s")

    @functools.partial(
        pl.kernel,
        mesh=mesh,
        out_type=jax.ShapeDtypeStruct((n_rows, dim), jnp.float32),
        scratch_types=[
            pltpu.VMEM((per_w,), jnp.int32),
            pltpu.VMEM((2, CHUNK, dim), jnp.float32),
            pltpu.SemaphoreType.DMA,
            pltpu.SemaphoreType.DMA,
            pltpu.SemaphoreType.DMA,
            pltpu.SemaphoreType.DMA,
        ],
        compiler_params=pltpu.CompilerParams(use_tc_tiling_on_sc=False),
    )
    def gather_kernel(x_hbm, table_hbm, out_hbm, idx_v, rows_v, sg0, sg1, so0, so1):
        wid = lax.axis_index("s") * info.num_cores + lax.axis_index("c")
        base = wid * per_w
        sg = (sg0, sg1)
        so = (so0, so1)

        pltpu.sync_copy(x_hbm.at[pl.ds(base, per_w)], idx_v)

        def gat(i, b):
            return pltpu.make_async_copy(
                table_hbm.at[idx_v.at[pl.ds(i * CHUNK, CHUNK)]],
                rows_v.at[b],
                sg[b],
            )

        def sto(i, b):
            return pltpu.make_async_copy(
                rows_v.at[b],
                out_hbm.at[pl.ds(base + i * CHUNK, CHUNK)],
                so[b],
            )

        gat(0, 0).start()

        def body(j, carry):
            i0 = 2 * j
            i1 = i0 + 1
            gat(i0, 0).wait()
            sto(i0, 0).start()

            @pl.when(j > 0)
            def _():
                sto(i0 - 1, 1).wait()

            gat(i1, 1).start()
            gat(i1, 1).wait()
            sto(i1, 1).start()

            @pl.when(j < n2 - 1)
            def _():
                sto(i0, 0).wait()
                gat(i0 + 2, 0).start()

            return carry

        lax.fori_loop(0, n2, body, 0)
        sto(n_chunks - 2, 0).wait()
        sto(n_chunks - 1, 1).wait()

    return gather_kernel


def kernel(x, table):
    n_rows = x.shape[0] * x.shape[1]
    vocab, dim = table.shape
    fn = _build(n_rows, vocab, dim)
    out = fn(x.reshape(-1).astype(jnp.int32), table)
    return out.reshape(x.shape + (dim,))
